# baseline (speedup 1.0000x reference)
import functools

import jax
import jax.numpy as jnp
from jax import lax
from jax.experimental import pallas as pl
from jax.experimental.pallas import tpu as pltpu
from jax.experimental.pallas import tpu_sc as plsc

D = 64
SEQ = 200
BATCH = 4096
NW = 32
CHUNK = 128
NTC = BATCH // CHUNK
NBUF = 4
LANES = 16


def _sc_body(xT_hbm, table_hbm, posp_hbm, out_hbm, *scratch):
    idx_v = scratch[0]
    pos_v = scratch[1]
    gbufs = scratch[2:2 + NBUF]
    sbufs = scratch[2 + NBUF:2 + 2 * NBUF]
    gsems = scratch[2 + 2 * NBUF:2 + 3 * NBUF]
    ssems = scratch[2 + 3 * NBUF:2 + 4 * NBUF]

    wid = lax.axis_index("c") * 16 + lax.axis_index("s")

    pltpu.sync_copy(xT_hbm.at[:, pl.ds(wid * CHUNK, CHUNK)], idx_v)
    pltpu.sync_copy(posp_hbm, pos_v.at[pl.ds(0, SEQ * D)])

    iota = lax.iota(jnp.int32, LANES)
    cvecs = [iota + (k * LANES) for k in range(CHUNK // LANES)]

    def gather(n, b):
        return pltpu.make_async_copy(
            table_hbm.at[idx_v.at[n]], gbufs[b], gsems[b])

    def scatter(n, b):
        return pltpu.make_async_copy(
            sbufs[b],
            out_hbm.at[pl.ds(n * (D // 8), D // 8), wid, :, :],
            ssems[b])

    for b in range(NBUF):
        gather(b, b).start()

    def outer(t, carry):
        for b in range(NBUF):
            n = t * NBUF + b
            gather(n, b).wait()

            @pl.when(t > 0)
            def _():
                scatter(n - NBUF, b).wait()

            pbase = lax.mul(n, D)

            def dloop(d, _, b=b, pbase=pbase):
                tr = lax.shift_right_logical(d, 3)
                rr = lax.bitwise_and(d, 7)
                pscal = pos_v[pl.ds(pbase + d, LANES)][0]
                dvec = jnp.broadcast_to(d, (LANES,)).astype(jnp.int32)
                for k in range(CHUNK // LANES):
                    val = plsc.load_gather(gbufs[b], [cvecs[k], dvec]) + pscal
                    sbufs[b][tr, rr, pl.ds(k * LANES, LANES)] = val
                return 0

            lax.fori_loop(0, D, dloop, 0)

            scatter(n, b).start()

            @pl.when(n + NBUF < SEQ)
            def _():
                gather(n + NBUF, b).start()
        return carry

    lax.fori_loop(0, SEQ // NBUF, outer, 0)

    for b in range(NBUF):
        scatter(SEQ - NBUF + b, b).wait()


_scratch = (
    [pltpu.VMEM((SEQ, CHUNK), jnp.int32),
     pltpu.VMEM((SEQ * D + LANES,), jnp.float32)]
    + [pltpu.VMEM((CHUNK, D), jnp.float32) for _ in range(NBUF)]
    + [pltpu.VMEM((D // 8, 8, CHUNK), jnp.float32) for _ in range(NBUF)]
    + [pltpu.SemaphoreType.DMA for _ in range(2 * NBUF)]
)

_sc_embed = functools.partial(
    pl.kernel,
    out_type=jax.ShapeDtypeStruct((SEQ * (D // 8), NTC, 8, CHUNK),
                                  jnp.float32),
    mesh=plsc.VectorSubcoreMesh(core_axis_name="c", subcore_axis_name="s"),
    scratch_types=_scratch,
    compiler_params=pltpu.CompilerParams(
        use_tc_tiling_on_sc=False, needs_layout_passes=False),
)(_sc_body)


def kernel(x, table, pos_embedding):
    B, N = x.shape
    xT = x.astype(jnp.int32).T
    posp = pos_embedding[0, :N, :].reshape(N * D)
    out4 = _sc_embed(xT, table, posp)
    return out4.transpose(1, 3, 0, 2).reshape(B, N, D)
